# baseline (speedup 1.0000x reference)
import functools

import jax
import jax.numpy as jnp
from jax.experimental import pallas as pl
from jax.experimental.pallas import tpu as pltpu


def _moe_dense_kernel(x_ref, wg_ref, bg_ref, w1_ref, b1_ref, w2_ref, b2_ref,
                      out_ref, ent_ref, gates_ref, *, n_tokens):
    t = pl.program_id(0)
    e = pl.program_id(1)

    @pl.when(e == 0)
    def _router():
        x = x_ref[...]
        logits = jnp.dot(x, wg_ref[...], preferred_element_type=jnp.float32)
        logits = logits + bg_ref[...]
        bt, ne = logits.shape
        idx = jax.lax.broadcasted_iota(jnp.int32, (bt, ne), 1)
        m1 = jnp.max(logits, axis=1, keepdims=True)
        is1 = logits == m1
        i1 = jnp.min(jnp.where(is1, idx, ne), axis=1, keepdims=True)
        oh1 = idx == i1
        masked = jnp.where(oh1, -jnp.inf, logits)
        m2 = jnp.max(masked, axis=1, keepdims=True)
        is2 = masked == m2
        i2 = jnp.min(jnp.where(is2, idx, ne), axis=1, keepdims=True)
        oh2 = idx == i2
        z = jnp.exp(m2 - m1)
        denom = 1.0 + z
        g1 = 1.0 / denom
        g2 = z / denom
        gates_ref[...] = g1 * oh1.astype(jnp.float32) + g2 * oh2.astype(jnp.float32)
        ent_tok = -(g1 * jnp.log(jnp.clip(g1, 1e-8, None))
                    + g2 * jnp.log(jnp.clip(g2, 1e-8, None)))
        part = (jnp.sum(ent_tok) / n_tokens).reshape(1, 1)
        prev = jnp.where(t == 0, jnp.zeros((1, 1), jnp.float32), ent_ref[...])
        ent_ref[...] = prev + part

    ne = gates_ref.shape[1]
    lane = jax.lax.broadcasted_iota(jnp.int32, (1, ne), 1)
    w = jnp.sum(gates_ref[...] * (lane == e).astype(jnp.float32),
                axis=1, keepdims=True)
    x = x_ref[...].astype(jnp.bfloat16)
    h = jnp.dot(x, w1_ref[0], preferred_element_type=jnp.float32)
    h = jnp.maximum(h + b1_ref[0], 0.0)
    y = jnp.dot(h.astype(jnp.bfloat16), w2_ref[0],
                preferred_element_type=jnp.float32)
    y = y + b2_ref[0]
    contrib = y * w

    @pl.when(e == 0)
    def _init():
        out_ref[...] = contrib

    @pl.when(e > 0)
    def _acc():
        out_ref[...] += contrib


def kernel(x, W1, b1, W2, b2, Wg, bg):
    B, N, D = x.shape
    E, _, DFF = W1.shape
    xf = x.reshape(N, D)
    BT = 512
    T = N // BT

    kern = functools.partial(_moe_dense_kernel, n_tokens=N)
    out, ent = pl.pallas_call(
        kern,
        grid=(T, E),
        in_specs=[
            pl.BlockSpec((BT, D), lambda t, e: (t, 0)),
            pl.BlockSpec((D, E), lambda t, e: (0, 0)),
            pl.BlockSpec((1, E), lambda t, e: (0, 0)),
            pl.BlockSpec((1, D, DFF), lambda t, e: (e, 0, 0)),
            pl.BlockSpec((1, 1, DFF), lambda t, e: (e, 0, 0)),
            pl.BlockSpec((1, DFF, D), lambda t, e: (e, 0, 0)),
            pl.BlockSpec((1, 1, D), lambda t, e: (e, 0, 0)),
        ],
        out_specs=[
            pl.BlockSpec((BT, D), lambda t, e: (t, 0)),
            pl.BlockSpec((1, 1), lambda t, e: (0, 0)),
        ],
        out_shape=[
            jax.ShapeDtypeStruct((N, D), jnp.float32),
            jax.ShapeDtypeStruct((1, 1), jnp.float32),
        ],
        scratch_shapes=[pltpu.VMEM((BT, E), jnp.float32)],
    )(xf, Wg, bg.reshape(1, E), W1.astype(jnp.bfloat16),
      b1.reshape(E, 1, DFF), W2.astype(jnp.bfloat16), b2.reshape(E, 1, D))

    return out.reshape(B, N, D), ent[0, 0]


# speedup vs baseline: 1.2516x; 1.2516x over previous
import functools

import jax
import jax.numpy as jnp
from jax.experimental import pallas as pl
from jax.experimental.pallas import tpu as pltpu


def _moe_dense_kernel(x_ref, wg_ref, bg_ref, w1_ref, b1_ref, w2_ref, b2_ref,
                      out_ref, ent_ref, gates_ref, *, bt):
    e = pl.program_id(0)
    t = pl.program_id(1)
    n_tokens = x_ref.shape[0]
    row = pl.ds(t * bt, bt)
    xt = x_ref[row, :]

    @pl.when(e == 0)
    def _router():
        logits = jnp.dot(xt, wg_ref[...], preferred_element_type=jnp.float32)
        logits = logits + bg_ref[...]
        ne = logits.shape[1]
        idx = jax.lax.broadcasted_iota(jnp.int32, (bt, ne), 1)
        m1 = jnp.max(logits, axis=1, keepdims=True)
        i1 = jnp.min(jnp.where(logits == m1, idx, ne), axis=1, keepdims=True)
        oh1 = idx == i1
        masked = jnp.where(oh1, -jnp.inf, logits)
        m2 = jnp.max(masked, axis=1, keepdims=True)
        i2 = jnp.min(jnp.where(masked == m2, idx, ne), axis=1, keepdims=True)
        oh2 = idx == i2
        z = jnp.exp(m2 - m1)
        denom = 1.0 + z
        g1 = 1.0 / denom
        g2 = z / denom
        gates_ref[row, :] = (g1 * oh1.astype(jnp.float32)
                             + g2 * oh2.astype(jnp.float32))
        ent_tok = -(g1 * jnp.log(jnp.clip(g1, 1e-8, None))
                    + g2 * jnp.log(jnp.clip(g2, 1e-8, None)))
        part = (jnp.sum(ent_tok) / n_tokens).reshape(1, 1)
        prev = jnp.where(t == 0, jnp.zeros((1, 1), jnp.float32), ent_ref[...])
        ent_ref[...] = prev + part

    ne = gates_ref.shape[1]
    lane = jax.lax.broadcasted_iota(jnp.int32, (1, ne), 1)
    w = jnp.sum(gates_ref[row, :] * (lane == e).astype(jnp.float32),
                axis=1, keepdims=True)
    h = jnp.dot(xt, w1_ref[0], preferred_element_type=jnp.float32)
    h = jnp.maximum(h + b1_ref[0], 0.0)
    y = jnp.dot(h, w2_ref[0], preferred_element_type=jnp.float32)
    y = y + b2_ref[0]
    contrib = y * w

    @pl.when(e == 0)
    def _init():
        out_ref[row, :] = contrib

    @pl.when(e > 0)
    def _acc():
        out_ref[row, :] += contrib


def kernel(x, W1, b1, W2, b2, Wg, bg):
    B, N, D = x.shape
    E, _, DFF = W1.shape
    xf = x.reshape(N, D)
    BT = 512
    T = N // BT

    kern = functools.partial(_moe_dense_kernel, bt=BT)
    out, ent = pl.pallas_call(
        kern,
        grid=(E, T),
        in_specs=[
            pl.BlockSpec((N, D), lambda e, t: (0, 0)),
            pl.BlockSpec((D, E), lambda e, t: (0, 0)),
            pl.BlockSpec((1, E), lambda e, t: (0, 0)),
            pl.BlockSpec((1, D, DFF), lambda e, t: (e, 0, 0)),
            pl.BlockSpec((1, 1, DFF), lambda e, t: (e, 0, 0)),
            pl.BlockSpec((1, DFF, D), lambda e, t: (e, 0, 0)),
            pl.BlockSpec((1, 1, D), lambda e, t: (e, 0, 0)),
        ],
        out_specs=[
            pl.BlockSpec((N, D), lambda e, t: (0, 0)),
            pl.BlockSpec((1, 1), lambda e, t: (0, 0)),
        ],
        out_shape=[
            jax.ShapeDtypeStruct((N, D), jnp.float32),
            jax.ShapeDtypeStruct((1, 1), jnp.float32),
        ],
        scratch_shapes=[pltpu.VMEM((N, E), jnp.float32)],
    )(xf, Wg, bg.reshape(1, E), W1, b1.reshape(E, 1, DFF),
      W2, b2.reshape(E, 1, D))

    return out.reshape(B, N, D), ent[0, 0]


# speedup vs baseline: 1.4293x; 1.1420x over previous
import functools

import jax
import jax.numpy as jnp
from jax import lax
from jax.experimental import pallas as pl
from jax.experimental.pallas import tpu as pltpu
from jax.experimental.pallas import tpu_sc as plsc

_BT = 256
_NSLOT = 6144
_NTILES = _NSLOT // _BT
_NW = 32
_LANES = 16


def _router_kernel(x_ref, wg_ref, bg_ref,
                   pos0_ref, pos1_ref, g1_ref, g2_ref, te_ref, ent_ref):
    n, _ = x_ref.shape
    ne = wg_ref.shape[1]
    logits = jnp.dot(x_ref[...], wg_ref[...],
                     preferred_element_type=jnp.float32) + bg_ref[...]
    idx = lax.broadcasted_iota(jnp.int32, (n, ne), 1)
    m1 = jnp.max(logits, axis=1, keepdims=True)
    i1 = jnp.min(jnp.where(logits == m1, idx, ne), axis=1, keepdims=True)
    oh1 = (idx == i1).astype(jnp.float32)
    masked = jnp.where(idx == i1, -jnp.inf, logits)
    m2 = jnp.max(masked, axis=1, keepdims=True)
    i2 = jnp.min(jnp.where(masked == m2, idx, ne), axis=1, keepdims=True)
    oh2 = (idx == i2).astype(jnp.float32)

    z = jnp.exp(m2 - m1)
    denom = 1.0 + z
    g1 = 1.0 / denom
    g2 = z / denom
    g1_ref[...] = g1
    g2_ref[...] = g2
    ent_tok = -(g1 * jnp.log(jnp.clip(g1, 1e-8, None))
                + g2 * jnp.log(jnp.clip(g2, 1e-8, None)))
    ent_ref[...] = (jnp.sum(ent_tok) / n).reshape(1, 1)

    ri = lax.broadcasted_iota(jnp.int32, (n, n), 0)
    ci = lax.broadcasted_iota(jnp.int32, (n, n), 1)
    ltri = (ci < ri).astype(jnp.float32)
    ohcat = jnp.concatenate([oh1, oh2], axis=1)
    csum = jnp.dot(ltri, ohcat, preferred_element_type=jnp.float32)
    c0 = csum[:, :ne]
    c1 = csum[:, ne:]

    cnt0 = jnp.sum(oh1, axis=0, keepdims=True)
    cnt1 = jnp.sum(oh2, axis=0, keepdims=True)
    counts = cnt0 + cnt1
    pcnt = jnp.floor((counts + (_BT - 1)) / _BT) * _BT
    eri = lax.broadcasted_iota(jnp.int32, (ne, ne), 0)
    eci = lax.broadcasted_iota(jnp.int32, (ne, ne), 1)
    estri = (eri < eci).astype(jnp.float32)
    offs = jnp.dot(pcnt, estri, preferred_element_type=jnp.float32)

    pos0 = jnp.sum(oh1 * (offs + c0), axis=1, keepdims=True)
    pos1 = jnp.sum(oh2 * (offs + cnt0 + c1), axis=1, keepdims=True)
    pos0_ref[...] = pos0.astype(jnp.int32)
    pos1_ref[...] = pos1.astype(jnp.int32)

    nt = te_ref.shape[0]
    jrow = (lax.broadcasted_iota(jnp.int32, (nt, ne), 0) * _BT).astype(jnp.float32)
    offs_b = jnp.broadcast_to(offs, (nt, ne))
    te = jnp.sum((offs_b <= jrow).astype(jnp.float32), axis=1, keepdims=True) - 1.0
    te_ref[...] = te.astype(jnp.int32)
    total = jnp.sum(pcnt).reshape(1, 1)
    te_ref[pl.ds(nt - 1, 1), :] = total.astype(jnp.int32)


def _run_router(xf, Wg, bg):
    n, d = xf.shape
    ne = Wg.shape[1]
    return pl.pallas_call(
        _router_kernel,
        out_shape=[
            jax.ShapeDtypeStruct((n, 1), jnp.int32),
            jax.ShapeDtypeStruct((n, 1), jnp.int32),
            jax.ShapeDtypeStruct((n, 1), jnp.float32),
            jax.ShapeDtypeStruct((n, 1), jnp.float32),
            jax.ShapeDtypeStruct((_NTILES + 1, 1), jnp.int32),
            jax.ShapeDtypeStruct((1, 1), jnp.float32),
        ],
    )(xf, Wg, bg.reshape(1, ne))


def _scatter_sc(xf, pos0, pos1):
    n, d = xf.shape
    chunk = n // _NW
    mesh = plsc.VectorSubcoreMesh(core_axis_name="c", subcore_axis_name="s")

    @functools.partial(
        pl.kernel, mesh=mesh,
        out_type=jax.ShapeDtypeStruct((_NSLOT, d), jnp.float32),
        scratch_types=[
            pltpu.VMEM((chunk, d), jnp.float32),
            pltpu.VMEM((chunk,), jnp.int32),
            pltpu.VMEM((chunk,), jnp.int32),
            pltpu.SemaphoreType.DMA,
        ],
    )
    def k(x_hbm, p0_hbm, p1_hbm, xs_hbm, xv, i0, i1, sem):
        wid = lax.axis_index("s") * 2 + lax.axis_index("c")
        base = wid * chunk
        pltpu.sync_copy(x_hbm.at[pl.ds(base, chunk)], xv)
        pltpu.sync_copy(p0_hbm.at[pl.ds(base, chunk)], i0)
        pltpu.sync_copy(p1_hbm.at[pl.ds(base, chunk)], i1)
        d1 = pltpu.async_copy(xv, xs_hbm.at[i0], sem)
        d2 = pltpu.async_copy(xv, xs_hbm.at[i1], sem)
        d1.wait()
        d2.wait()

    return k(xf, pos0, pos1)


def _ffn_kernel(sp_ref, xs_ref, w1_ref, b1_ref, w2_ref, b2_ref, y_ref):
    j = pl.program_id(0)

    @pl.when(j * _BT < sp_ref[_NTILES])
    def _():
        h = jnp.dot(xs_ref[...], w1_ref[0], preferred_element_type=jnp.float32)
        h = jnp.maximum(h + b1_ref[0], 0.0)
        y = jnp.dot(h, w2_ref[0], preferred_element_type=jnp.float32)
        y_ref[...] = y + b2_ref[0]


def _run_ffn(sp, xs, W1, b1, W2, b2):
    e, d, dff = W1.shape
    grid_spec = pltpu.PrefetchScalarGridSpec(
        num_scalar_prefetch=1,
        grid=(_NTILES,),
        in_specs=[
            pl.BlockSpec((_BT, d), lambda j, sp: (j, 0)),
            pl.BlockSpec((1, d, dff), lambda j, sp: (sp[j], 0, 0)),
            pl.BlockSpec((1, 1, dff), lambda j, sp: (sp[j], 0, 0)),
            pl.BlockSpec((1, dff, d), lambda j, sp: (sp[j], 0, 0)),
            pl.BlockSpec((1, 1, d), lambda j, sp: (sp[j], 0, 0)),
        ],
        out_specs=pl.BlockSpec((_BT, d), lambda j, sp: (j, 0)),
    )
    return pl.pallas_call(
        _ffn_kernel,
        grid_spec=grid_spec,
        out_shape=jax.ShapeDtypeStruct((_NSLOT, d), jnp.float32),
    )(sp, xs, W1, b1.reshape(e, 1, dff), W2, b2.reshape(e, 1, d))


def _combine_sc(y, pos0, pos1, g1, g2):
    nslot, d = y.shape
    n = pos0.shape[0]
    chunk = n // _NW
    half = chunk // 2
    mesh = plsc.VectorSubcoreMesh(core_axis_name="c", subcore_axis_name="s")

    @functools.partial(
        pl.kernel, mesh=mesh,
        out_type=jax.ShapeDtypeStruct((n, d), jnp.float32),
        scratch_types=[
            pltpu.VMEM((half, d), jnp.float32),
            pltpu.VMEM((half, d), jnp.float32),
            pltpu.VMEM((half,), jnp.int32),
            pltpu.VMEM((half,), jnp.int32),
            pltpu.VMEM((half,), jnp.float32),
            pltpu.VMEM((half,), jnp.float32),
            pltpu.SemaphoreType.DMA,
        ],
    )
    def k(y_hbm, p0_hbm, p1_hbm, g1_hbm, g2_hbm, out_hbm,
          v0, v1, i0, i1, gc0, gc1, sem):
        wid = lax.axis_index("s") * 2 + lax.axis_index("c")
        base = wid * chunk
        for hh in range(2):
            hbase = base + hh * half
            pltpu.sync_copy(p0_hbm.at[pl.ds(hbase, half)], i0)
            pltpu.sync_copy(p1_hbm.at[pl.ds(hbase, half)], i1)
            pltpu.sync_copy(g1_hbm.at[pl.ds(hbase, half)], gc0)
            pltpu.sync_copy(g2_hbm.at[pl.ds(hbase, half)], gc1)
            da = pltpu.async_copy(y_hbm.at[i0], v0, sem)
            db = pltpu.async_copy(y_hbm.at[i1], v1, sem)
            da.wait()
            db.wait()

            def body(r, _):
                grp = pl.ds((r // _LANES) * _LANES, _LANES)
                bidx = jnp.full((_LANES,), r % _LANES, jnp.int32)
                dn = lax.GatherDimensionNumbers(
                    offset_dims=(), collapsed_slice_dims=(0,),
                    start_index_map=(0,))
                s0 = lax.gather(gc0[grp], bidx[:, None], dn, (1,),
                                mode=lax.GatherScatterMode.PROMISE_IN_BOUNDS)
                s1 = lax.gather(gc1[grp], bidx[:, None], dn, (1,),
                                mode=lax.GatherScatterMode.PROMISE_IN_BOUNDS)
                for c in range(d // _LANES):
                    sl = pl.ds(c * _LANES, _LANES)
                    v0[r, sl] = v0[r, sl] * s0 + v1[r, sl] * s1
                return 0

            lax.fori_loop(0, half, body, 0)
            pltpu.sync_copy(v0, out_hbm.at[pl.ds(hbase, half)])

    return k(y, pos0, pos1, g1, g2)


def kernel(x, W1, b1, W2, b2, Wg, bg):
    B, N, D = x.shape
    E, _, DFF = W1.shape
    xf = x.reshape(N, D)

    pos0, pos1, g1, g2, te, ent = _run_router(xf, Wg, bg)
    pos0 = pos0.reshape(N)
    pos1 = pos1.reshape(N)
    xs = _scatter_sc(xf, pos0, pos1)
    y = _run_ffn(te.reshape(_NTILES + 1), xs, W1, b1, W2, b2)
    out = _combine_sc(y, pos0, pos1, g1.reshape(N), g2.reshape(N))
    return out.reshape(B, N, D), ent[0, 0]


# speedup vs baseline: 1.4675x; 1.0268x over previous
import functools

import jax
import jax.numpy as jnp
from jax import lax
from jax.experimental import pallas as pl
from jax.experimental.pallas import tpu as pltpu
from jax.experimental.pallas import tpu_sc as plsc

_BT = 256
_NSLOT = 6144
_NTILES = _NSLOT // _BT
_NW = 32
_LANES = 16
_GW = 128


def _router_kernel(x_ref, wg_ref, bg_ref,
                   pos0_ref, pos1_ref, g1b_ref, g2b_ref, te_ref, ent_ref):
    n, _ = x_ref.shape
    ne = wg_ref.shape[1]
    logits = jnp.dot(x_ref[...], wg_ref[...],
                     preferred_element_type=jnp.float32) + bg_ref[...]
    idx = lax.broadcasted_iota(jnp.int32, (n, ne), 1)
    m1 = jnp.max(logits, axis=1, keepdims=True)
    i1 = jnp.min(jnp.where(logits == m1, idx, ne), axis=1, keepdims=True)
    oh1 = (idx == i1).astype(jnp.float32)
    masked = jnp.where(idx == i1, -jnp.inf, logits)
    m2 = jnp.max(masked, axis=1, keepdims=True)
    i2 = jnp.min(jnp.where(masked == m2, idx, ne), axis=1, keepdims=True)
    oh2 = (idx == i2).astype(jnp.float32)

    z = jnp.exp(m2 - m1)
    denom = 1.0 + z
    g1 = 1.0 / denom
    g2 = z / denom
    g1b_ref[...] = jnp.broadcast_to(g1, (n, _GW))
    g2b_ref[...] = jnp.broadcast_to(g2, (n, _GW))
    ent_tok = -(g1 * jnp.log(jnp.clip(g1, 1e-8, None))
                + g2 * jnp.log(jnp.clip(g2, 1e-8, None)))
    ent_ref[...] = (jnp.sum(ent_tok) / n).reshape(1, 1)

    ch = 256
    ri = lax.broadcasted_iota(jnp.int32, (ch, ch), 0)
    ci = lax.broadcasted_iota(jnp.int32, (ch, ch), 1)
    ltri = (ci < ri).astype(jnp.float32)
    ohcat = jnp.concatenate([oh1, oh2], axis=1)
    carry = jnp.zeros((1, 2 * ne), jnp.float32)
    chunks = []
    for c in range(n // ch):
        ohc = ohcat[c * ch:(c + 1) * ch, :]
        chunks.append(jnp.dot(ltri, ohc, preferred_element_type=jnp.float32)
                      + carry)
        carry = carry + jnp.sum(ohc, axis=0, keepdims=True)
    csum = jnp.concatenate(chunks, axis=0)
    c0 = csum[:, :ne]
    c1 = csum[:, ne:]

    cnt0 = jnp.sum(oh1, axis=0, keepdims=True)
    cnt1 = jnp.sum(oh2, axis=0, keepdims=True)
    counts = cnt0 + cnt1
    pcnt = jnp.floor((counts + (_BT - 1)) / _BT) * _BT
    eri = lax.broadcasted_iota(jnp.int32, (ne, ne), 0)
    eci = lax.broadcasted_iota(jnp.int32, (ne, ne), 1)
    estri = (eri < eci).astype(jnp.float32)
    offs = jnp.dot(pcnt, estri, preferred_element_type=jnp.float32)

    pos0 = jnp.sum(oh1 * (offs + c0), axis=1, keepdims=True)
    pos1 = jnp.sum(oh2 * (offs + cnt0 + c1), axis=1, keepdims=True)
    pos0_ref[...] = pos0.astype(jnp.int32)
    pos1_ref[...] = pos1.astype(jnp.int32)

    nt = te_ref.shape[0]
    jrow = (lax.broadcasted_iota(jnp.int32, (nt, ne), 0) * _BT).astype(jnp.float32)
    offs_b = jnp.broadcast_to(offs, (nt, ne))
    te = jnp.sum((offs_b <= jrow).astype(jnp.float32), axis=1, keepdims=True) - 1.0
    te_ref[...] = te.astype(jnp.int32)
    total = jnp.sum(pcnt).reshape(1, 1)
    te_ref[pl.ds(nt - 1, 1), :] = total.astype(jnp.int32)


def _run_router(xf, Wg, bg):
    n, d = xf.shape
    ne = Wg.shape[1]
    return pl.pallas_call(
        _router_kernel,
        out_shape=[
            jax.ShapeDtypeStruct((n, 1), jnp.int32),
            jax.ShapeDtypeStruct((n, 1), jnp.int32),
            jax.ShapeDtypeStruct((n, _GW), jnp.float32),
            jax.ShapeDtypeStruct((n, _GW), jnp.float32),
            jax.ShapeDtypeStruct((_NTILES + 1, 1), jnp.int32),
            jax.ShapeDtypeStruct((1, 1), jnp.float32),
        ],
    )(xf, Wg, bg.reshape(1, ne))


def _scatter_sc(xf, pos0, pos1, g1b, g2b):
    n, d = xf.shape
    chunk = n // _NW
    mesh = plsc.VectorSubcoreMesh(core_axis_name="c", subcore_axis_name="s")

    @functools.partial(
        pl.kernel, mesh=mesh,
        out_type=[
            jax.ShapeDtypeStruct((_NSLOT, d), jnp.float32),
            jax.ShapeDtypeStruct((_NSLOT, _GW), jnp.float32),
        ],
        scratch_types=[
            pltpu.VMEM((chunk, d), jnp.float32),
            pltpu.VMEM((chunk, _GW), jnp.float32),
            pltpu.VMEM((chunk, _GW), jnp.float32),
            pltpu.VMEM((chunk,), jnp.int32),
            pltpu.VMEM((chunk,), jnp.int32),
            pltpu.SemaphoreType.DMA,
        ],
    )
    def k(x_hbm, p0_hbm, p1_hbm, g1_hbm, g2_hbm, xs_hbm, gs_hbm,
          xv, gb0, gb1, i0, i1, sem):
        wid = lax.axis_index("s") * 2 + lax.axis_index("c")
        base = wid * chunk
        pltpu.sync_copy(x_hbm.at[pl.ds(base, chunk)], xv)
        pltpu.sync_copy(p0_hbm.at[pl.ds(base, chunk)], i0)
        pltpu.sync_copy(p1_hbm.at[pl.ds(base, chunk)], i1)
        pltpu.sync_copy(g1_hbm.at[pl.ds(base, chunk)], gb0)
        pltpu.sync_copy(g2_hbm.at[pl.ds(base, chunk)], gb1)
        d1 = pltpu.async_copy(xv, xs_hbm.at[i0], sem)
        d2 = pltpu.async_copy(xv, xs_hbm.at[i1], sem)
        d3 = pltpu.async_copy(gb0, gs_hbm.at[i0], sem)
        d4 = pltpu.async_copy(gb1, gs_hbm.at[i1], sem)
        d1.wait()
        d2.wait()
        d3.wait()
        d4.wait()

    return k(xf, pos0, pos1, g1b, g2b)


def _ffn_kernel(sp_ref, xs_ref, gs_ref, w1_ref, b1_ref, w2_ref, b2_ref, y_ref):
    j = pl.program_id(0)

    @pl.when(j * _BT < sp_ref[_NTILES])
    def _():
        h = jnp.dot(xs_ref[...], w1_ref[0], preferred_element_type=jnp.float32)
        h = jnp.maximum(h + b1_ref[0], 0.0)
        y = jnp.dot(h, w2_ref[0], preferred_element_type=jnp.float32)
        y_ref[...] = (y + b2_ref[0]) * gs_ref[:, 0:1]


def _run_ffn(sp, xs, gs, W1, b1, W2, b2):
    e, d, dff = W1.shape
    grid_spec = pltpu.PrefetchScalarGridSpec(
        num_scalar_prefetch=1,
        grid=(_NTILES,),
        in_specs=[
            pl.BlockSpec((_BT, d), lambda j, sp: (j, 0)),
            pl.BlockSpec((_BT, _GW), lambda j, sp: (j, 0)),
            pl.BlockSpec((1, d, dff), lambda j, sp: (sp[j], 0, 0)),
            pl.BlockSpec((1, 1, dff), lambda j, sp: (sp[j], 0, 0)),
            pl.BlockSpec((1, dff, d), lambda j, sp: (sp[j], 0, 0)),
            pl.BlockSpec((1, 1, d), lambda j, sp: (sp[j], 0, 0)),
        ],
        out_specs=pl.BlockSpec((_BT, d), lambda j, sp: (j, 0)),
    )
    return pl.pallas_call(
        _ffn_kernel,
        grid_spec=grid_spec,
        out_shape=jax.ShapeDtypeStruct((_NSLOT, d), jnp.float32),
    )(sp, xs, gs, W1, b1.reshape(e, 1, dff), W2, b2.reshape(e, 1, d))


def _combine_sc(y, pos0, pos1):
    nslot, d = y.shape
    n = pos0.shape[0]
    chunk = n // _NW
    half = chunk // 2
    mesh = plsc.VectorSubcoreMesh(core_axis_name="c", subcore_axis_name="s")

    @functools.partial(
        pl.kernel, mesh=mesh,
        out_type=jax.ShapeDtypeStruct((n, d), jnp.float32),
        scratch_types=[
            pltpu.VMEM((half, d), jnp.float32),
            pltpu.VMEM((half, d), jnp.float32),
            pltpu.VMEM((half,), jnp.int32),
            pltpu.VMEM((half,), jnp.int32),
            pltpu.SemaphoreType.DMA,
        ],
    )
    def k(y_hbm, p0_hbm, p1_hbm, out_hbm, v0, v1, i0, i1, sem):
        wid = lax.axis_index("s") * 2 + lax.axis_index("c")
        base = wid * chunk
        for hh in range(2):
            hbase = base + hh * half
            pltpu.sync_copy(p0_hbm.at[pl.ds(hbase, half)], i0)
            pltpu.sync_copy(p1_hbm.at[pl.ds(hbase, half)], i1)
            da = pltpu.async_copy(y_hbm.at[i0], v0, sem)
            db = pltpu.async_copy(y_hbm.at[i1], v1, sem)
            da.wait()
            db.wait()

            def body(r, _):
                for c in range(d // _LANES):
                    sl = pl.ds(c * _LANES, _LANES)
                    v0[r, sl] = v0[r, sl] + v1[r, sl]
                return 0

            lax.fori_loop(0, half, body, 0)
            pltpu.sync_copy(v0, out_hbm.at[pl.ds(hbase, half)])

    return k(y, pos0, pos1)


def kernel(x, W1, b1, W2, b2, Wg, bg):
    B, N, D = x.shape
    E, _, DFF = W1.shape
    xf = x.reshape(N, D)

    pos0, pos1, g1b, g2b, te, ent = _run_router(xf, Wg, bg)
    pos0 = pos0.reshape(N)
    pos1 = pos1.reshape(N)
    xs, gs = _scatter_sc(xf, pos0, pos1, g1b, g2b)
    y = _run_ffn(te.reshape(_NTILES + 1), xs, gs, W1, b1, W2, b2)
    out = _combine_sc(y, pos0, pos1)
    return out.reshape(B, N, D), ent[0, 0]


# speedup vs baseline: 1.5728x; 1.0718x over previous
import functools

import jax
import jax.numpy as jnp
from jax import lax
from jax.experimental import pallas as pl
from jax.experimental.pallas import tpu as pltpu
from jax.experimental.pallas import tpu_sc as plsc

_BT = 512
_NSLOT = 8192
_NTILES = _NSLOT // _BT
_NW = 32
_LANES = 16
_GW = 128


def _router_kernel(x_ref, wg_ref, bg_ref,
                   pos0_ref, pos1_ref, g1b_ref, g2b_ref, te_ref, ent_ref):
    n, _ = x_ref.shape
    ne = wg_ref.shape[1]
    logits = jnp.dot(x_ref[...], wg_ref[...],
                     preferred_element_type=jnp.float32) + bg_ref[...]
    idx = lax.broadcasted_iota(jnp.int32, (n, ne), 1)
    m1 = jnp.max(logits, axis=1, keepdims=True)
    i1 = jnp.min(jnp.where(logits == m1, idx, ne), axis=1, keepdims=True)
    oh1 = (idx == i1).astype(jnp.float32)
    masked = jnp.where(idx == i1, -jnp.inf, logits)
    m2 = jnp.max(masked, axis=1, keepdims=True)
    i2 = jnp.min(jnp.where(masked == m2, idx, ne), axis=1, keepdims=True)
    oh2 = (idx == i2).astype(jnp.float32)

    z = jnp.exp(m2 - m1)
    denom = 1.0 + z
    g1 = 1.0 / denom
    g2 = z / denom
    g1b_ref[...] = jnp.broadcast_to(g1, (n, _GW))
    g2b_ref[...] = jnp.broadcast_to(g2, (n, _GW))
    ent_tok = -(g1 * jnp.log(jnp.clip(g1, 1e-8, None))
                + g2 * jnp.log(jnp.clip(g2, 1e-8, None)))
    ent_ref[...] = (jnp.sum(ent_tok) / n).reshape(1, 1)

    ch = 256
    ri = lax.broadcasted_iota(jnp.int32, (ch, ch), 0)
    ci = lax.broadcasted_iota(jnp.int32, (ch, ch), 1)
    ltri = (ci < ri).astype(jnp.float32)
    ohcat = jnp.concatenate([oh1, oh2], axis=1)
    carry = jnp.zeros((1, 2 * ne), jnp.float32)
    chunks = []
    for c in range(n // ch):
        ohc = ohcat[c * ch:(c + 1) * ch, :]
        chunks.append(jnp.dot(ltri, ohc, preferred_element_type=jnp.float32)
                      + carry)
        carry = carry + jnp.sum(ohc, axis=0, keepdims=True)
    csum = jnp.concatenate(chunks, axis=0)
    c0 = csum[:, :ne]
    c1 = csum[:, ne:]

    cnt0 = jnp.sum(oh1, axis=0, keepdims=True)
    cnt1 = jnp.sum(oh2, axis=0, keepdims=True)
    counts = cnt0 + cnt1
    pcnt = jnp.floor((counts + (_BT - 1)) / _BT) * _BT
    eri = lax.broadcasted_iota(jnp.int32, (ne, ne), 0)
    eci = lax.broadcasted_iota(jnp.int32, (ne, ne), 1)
    estri = (eri < eci).astype(jnp.float32)
    offs = jnp.dot(pcnt, estri, preferred_element_type=jnp.float32)

    pos0 = jnp.sum(oh1 * (offs + c0), axis=1, keepdims=True)
    pos1 = jnp.sum(oh2 * (offs + cnt0 + c1), axis=1, keepdims=True)
    pos0_ref[...] = pos0.astype(jnp.int32)
    pos1_ref[...] = pos1.astype(jnp.int32)

    nt = te_ref.shape[0]
    jrow = (lax.broadcasted_iota(jnp.int32, (nt, ne), 0) * _BT).astype(jnp.float32)
    offs_b = jnp.broadcast_to(offs, (nt, ne))
    te = jnp.sum((offs_b <= jrow).astype(jnp.float32), axis=1, keepdims=True) - 1.0
    te_ref[...] = te.astype(jnp.int32)
    total = jnp.sum(pcnt).reshape(1, 1)
    te_ref[pl.ds(nt - 1, 1), :] = total.astype(jnp.int32)


def _run_router(xf, Wg, bg):
    n, d = xf.shape
    ne = Wg.shape[1]
    return pl.pallas_call(
        _router_kernel,
        out_shape=[
            jax.ShapeDtypeStruct((n, 1), jnp.int32),
            jax.ShapeDtypeStruct((n, 1), jnp.int32),
            jax.ShapeDtypeStruct((n, _GW), jnp.float32),
            jax.ShapeDtypeStruct((n, _GW), jnp.float32),
            jax.ShapeDtypeStruct((_NTILES + 1, 1), jnp.int32),
            jax.ShapeDtypeStruct((1, 1), jnp.float32),
        ],
    )(xf, Wg, bg.reshape(1, ne))


def _scatter_sc(xf, pos0, pos1, g1b, g2b):
    n, d = xf.shape
    chunk = n // _NW
    mesh = plsc.VectorSubcoreMesh(core_axis_name="c", subcore_axis_name="s")

    @functools.partial(
        pl.kernel, mesh=mesh,
        out_type=[
            jax.ShapeDtypeStruct((_NSLOT, d), jnp.float32),
            jax.ShapeDtypeStruct((_NSLOT, _GW), jnp.float32),
        ],
        scratch_types=[
            pltpu.VMEM((chunk, d), jnp.float32),
            pltpu.VMEM((chunk, _GW), jnp.float32),
            pltpu.VMEM((chunk, _GW), jnp.float32),
            pltpu.VMEM((chunk,), jnp.int32),
            pltpu.VMEM((chunk,), jnp.int32),
            pltpu.SemaphoreType.DMA,
        ],
    )
    def k(x_hbm, p0_hbm, p1_hbm, g1_hbm, g2_hbm, xs_hbm, gs_hbm,
          xv, gb0, gb1, i0, i1, sem):
        wid = lax.axis_index("s") * 2 + lax.axis_index("c")
        base = wid * chunk
        pltpu.sync_copy(x_hbm.at[pl.ds(base, chunk)], xv)
        pltpu.sync_copy(p0_hbm.at[pl.ds(base, chunk)], i0)
        pltpu.sync_copy(p1_hbm.at[pl.ds(base, chunk)], i1)
        pltpu.sync_copy(g1_hbm.at[pl.ds(base, chunk)], gb0)
        pltpu.sync_copy(g2_hbm.at[pl.ds(base, chunk)], gb1)
        d1 = pltpu.async_copy(xv, xs_hbm.at[i0], sem)
        d2 = pltpu.async_copy(xv, xs_hbm.at[i1], sem)
        d3 = pltpu.async_copy(gb0, gs_hbm.at[i0], sem)
        d4 = pltpu.async_copy(gb1, gs_hbm.at[i1], sem)
        d1.wait()
        d2.wait()
        d3.wait()
        d4.wait()

    return k(xf, pos0, pos1, g1b, g2b)


def _ffn_kernel(sp_ref, xs_ref, gs_ref, w1_ref, b1_ref, w2_ref, b2_ref, y_ref):
    j = pl.program_id(0)

    @pl.when(j * _BT < sp_ref[_NTILES])
    def _():
        h = jnp.dot(xs_ref[...], w1_ref[0], preferred_element_type=jnp.float32)
        h = jnp.maximum(h + b1_ref[0], 0.0)
        y = jnp.dot(h, w2_ref[0], preferred_element_type=jnp.float32)
        y_ref[...] = (y + b2_ref[0]) * gs_ref[:, 0:1]


def _run_ffn(sp, xs, gs, W1, b1, W2, b2):
    e, d, dff = W1.shape
    grid_spec = pltpu.PrefetchScalarGridSpec(
        num_scalar_prefetch=1,
        grid=(_NTILES,),
        in_specs=[
            pl.BlockSpec((_BT, d), lambda j, sp: (j, 0)),
            pl.BlockSpec((_BT, _GW), lambda j, sp: (j, 0)),
            pl.BlockSpec((1, d, dff), lambda j, sp: (sp[j], 0, 0)),
            pl.BlockSpec((1, 1, dff), lambda j, sp: (sp[j], 0, 0)),
            pl.BlockSpec((1, dff, d), lambda j, sp: (sp[j], 0, 0)),
            pl.BlockSpec((1, 1, d), lambda j, sp: (sp[j], 0, 0)),
        ],
        out_specs=pl.BlockSpec((_BT, d), lambda j, sp: (j, 0)),
    )
    return pl.pallas_call(
        _ffn_kernel,
        grid_spec=grid_spec,
        out_shape=jax.ShapeDtypeStruct((_NSLOT, d), jnp.float32),
    )(sp, xs, gs, W1, b1.reshape(e, 1, dff), W2, b2.reshape(e, 1, d))


def _combine_sc(y, pos0, pos1):
    nslot, d = y.shape
    n = pos0.shape[0]
    chunk = n // _NW
    half = chunk // 2
    mesh = plsc.VectorSubcoreMesh(core_axis_name="c", subcore_axis_name="s")

    @functools.partial(
        pl.kernel, mesh=mesh,
        out_type=jax.ShapeDtypeStruct((n, d), jnp.float32),
        scratch_types=[
            pltpu.VMEM((half, d), jnp.float32),
            pltpu.VMEM((half, d), jnp.float32),
            pltpu.VMEM((half,), jnp.int32),
            pltpu.VMEM((half,), jnp.int32),
            pltpu.SemaphoreType.DMA,
        ],
    )
    def k(y_hbm, p0_hbm, p1_hbm, out_hbm, v0, v1, i0, i1, sem):
        wid = lax.axis_index("s") * 2 + lax.axis_index("c")
        base = wid * chunk
        for hh in range(2):
            hbase = base + hh * half
            pltpu.sync_copy(p0_hbm.at[pl.ds(hbase, half)], i0)
            pltpu.sync_copy(p1_hbm.at[pl.ds(hbase, half)], i1)
            da = pltpu.async_copy(y_hbm.at[i0], v0, sem)
            db = pltpu.async_copy(y_hbm.at[i1], v1, sem)
            da.wait()
            db.wait()

            def body(r, _):
                for c in range(d // _LANES):
                    sl = pl.ds(c * _LANES, _LANES)
                    v0[r, sl] = v0[r, sl] + v1[r, sl]
                return 0

            lax.fori_loop(0, half, body, 0)
            pltpu.sync_copy(v0, out_hbm.at[pl.ds(hbase, half)])

    return k(y, pos0, pos1)


def kernel(x, W1, b1, W2, b2, Wg, bg):
    B, N, D = x.shape
    E, _, DFF = W1.shape
    xf = x.reshape(N, D)

    pos0, pos1, g1b, g2b, te, ent = _run_router(xf, Wg, bg)
    pos0 = pos0.reshape(N)
    pos1 = pos1.reshape(N)
    xs, gs = _scatter_sc(xf, pos0, pos1, g1b, g2b)
    y = _run_ffn(te.reshape(_NTILES + 1), xs, gs, W1, b1, W2, b2)
    out = _combine_sc(y, pos0, pos1)
    return out.reshape(B, N, D), ent[0, 0]


# speedup vs baseline: 1.6350x; 1.0395x over previous
import functools

import jax
import jax.numpy as jnp
from jax import lax
from jax.experimental import pallas as pl
from jax.experimental.pallas import tpu as pltpu
from jax.experimental.pallas import tpu_sc as plsc

_BT = 256
_NSLOT = 6144
_NTILES = _NSLOT // _BT
_NW = 32
_LANES = 16
_GW = 128


def _router_kernel(x_ref, wg_ref, bg_ref,
                   pos0_ref, pos1_ref, g1b_ref, g2b_ref, te_ref, ent_ref):
    n, _ = x_ref.shape
    ne = wg_ref.shape[1]
    logits = jnp.dot(x_ref[...], wg_ref[...],
                     preferred_element_type=jnp.float32) + bg_ref[...]
    idx = lax.broadcasted_iota(jnp.int32, (n, ne), 1)
    m1 = jnp.max(logits, axis=1, keepdims=True)
    i1 = jnp.min(jnp.where(logits == m1, idx, ne), axis=1, keepdims=True)
    oh1 = (idx == i1).astype(jnp.float32)
    masked = jnp.where(idx == i1, -jnp.inf, logits)
    m2 = jnp.max(masked, axis=1, keepdims=True)
    i2 = jnp.min(jnp.where(masked == m2, idx, ne), axis=1, keepdims=True)
    oh2 = (idx == i2).astype(jnp.float32)

    z = jnp.exp(m2 - m1)
    denom = 1.0 + z
    g1 = 1.0 / denom
    g2 = z / denom
    g1b_ref[...] = jnp.broadcast_to(g1, (n, _GW))
    g2b_ref[...] = jnp.broadcast_to(g2, (n, _GW))
    ent_tok = -(g1 * jnp.log(jnp.clip(g1, 1e-8, None))
                + g2 * jnp.log(jnp.clip(g2, 1e-8, None)))
    ent_ref[...] = (jnp.sum(ent_tok) / n).reshape(1, 1)

    ch = 256
    ri = lax.broadcasted_iota(jnp.int32, (ch, ch), 0)
    ci = lax.broadcasted_iota(jnp.int32, (ch, ch), 1)
    ltri = (ci < ri).astype(jnp.float32)
    ohcat = jnp.concatenate([oh1, oh2], axis=1)
    carry = jnp.zeros((1, 2 * ne), jnp.float32)
    chunks = []
    for c in range(n // ch):
        ohc = ohcat[c * ch:(c + 1) * ch, :]
        chunks.append(jnp.dot(ltri, ohc, preferred_element_type=jnp.float32)
                      + carry)
        carry = carry + jnp.sum(ohc, axis=0, keepdims=True)
    csum = jnp.concatenate(chunks, axis=0)
    c0 = csum[:, :ne]
    c1 = csum[:, ne:]

    cnt0 = jnp.sum(oh1, axis=0, keepdims=True)
    cnt1 = jnp.sum(oh2, axis=0, keepdims=True)
    counts = cnt0 + cnt1
    pcnt = jnp.floor((counts + (_BT - 1)) / _BT) * _BT
    eri = lax.broadcasted_iota(jnp.int32, (ne, ne), 0)
    eci = lax.broadcasted_iota(jnp.int32, (ne, ne), 1)
    estri = (eri < eci).astype(jnp.float32)
    offs = jnp.dot(pcnt, estri, preferred_element_type=jnp.float32)

    pos0 = jnp.sum(oh1 * (offs + c0), axis=1, keepdims=True)
    pos1 = jnp.sum(oh2 * (offs + cnt0 + c1), axis=1, keepdims=True)
    pos0_ref[...] = pos0.astype(jnp.int32)
    pos1_ref[...] = pos1.astype(jnp.int32)

    nt = _NTILES
    jrow = (lax.broadcasted_iota(jnp.int32, (nt, ne), 0) * _BT).astype(jnp.float32)
    offs_b = jnp.broadcast_to(offs, (nt, ne))
    te = jnp.sum((offs_b <= jrow).astype(jnp.float32), axis=1, keepdims=True) - 1.0

    te_shift = jnp.concatenate([jnp.full((1, 1), -1.0, jnp.float32),
                                te[:nt - 1]], axis=0)
    bnd = (te != te_shift).astype(jnp.float32)
    tri = lax.broadcasted_iota(jnp.int32, (nt, nt), 1) <= \
        lax.broadcasted_iota(jnp.int32, (nt, nt), 0)
    cumb = jnp.dot(tri.astype(jnp.float32), bnd,
                   preferred_element_type=jnp.float32)
    slot = (cumb - 1.0) - 2.0 * jnp.floor((cumb - 1.0) / 2.0)
    active_row = (pcnt > 0).astype(jnp.float32)
    m = (eci > eri) & (jnp.broadcast_to(active_row, (ne, ne)) > 0)
    nxt_e = jnp.min(jnp.where(m, eci.astype(jnp.float32), 99.0),
                    axis=1, keepdims=True)
    te_oh = (te == lax.broadcasted_iota(jnp.int32, (nt, ne), 1)
             .astype(jnp.float32)).astype(jnp.float32)
    nxt = jnp.dot(te_oh, nxt_e, preferred_element_type=jnp.float32)
    nxt = jnp.where(nxt == 99.0, te, nxt)

    total = jnp.sum(pcnt).reshape(1, 1)
    spx = jnp.concatenate([te, slot, bnd, nxt, total], axis=0)
    te_ref[...] = spx.astype(jnp.int32)


def _run_router(xf, Wg, bg):
    n, d = xf.shape
    ne = Wg.shape[1]
    return pl.pallas_call(
        _router_kernel,
        out_shape=[
            jax.ShapeDtypeStruct((n, 1), jnp.int32),
            jax.ShapeDtypeStruct((n, 1), jnp.int32),
            jax.ShapeDtypeStruct((n, _GW), jnp.float32),
            jax.ShapeDtypeStruct((n, _GW), jnp.float32),
            jax.ShapeDtypeStruct((4 * _NTILES + 1, 1), jnp.int32),
            jax.ShapeDtypeStruct((1, 1), jnp.float32),
        ],
    )(xf, Wg, bg.reshape(1, ne))


def _scatter_sc(xf, pos0, pos1, g1b, g2b):
    n, d = xf.shape
    chunk = n // _NW
    mesh = plsc.VectorSubcoreMesh(core_axis_name="c", subcore_axis_name="s")

    @functools.partial(
        pl.kernel, mesh=mesh,
        out_type=[
            jax.ShapeDtypeStruct((_NSLOT, d), jnp.float32),
            jax.ShapeDtypeStruct((_NSLOT, _GW), jnp.float32),
        ],
        scratch_types=[
            pltpu.VMEM((chunk, d), jnp.float32),
            pltpu.VMEM((chunk, _GW), jnp.float32),
            pltpu.VMEM((chunk, _GW), jnp.float32),
            pltpu.VMEM((chunk,), jnp.int32),
            pltpu.VMEM((chunk,), jnp.int32),
            pltpu.SemaphoreType.DMA,
        ],
    )
    def k(x_hbm, p0_hbm, p1_hbm, g1_hbm, g2_hbm, xs_hbm, gs_hbm,
          xv, gb0, gb1, i0, i1, sem):
        wid = lax.axis_index("s") * 2 + lax.axis_index("c")
        base = wid * chunk
        pltpu.sync_copy(x_hbm.at[pl.ds(base, chunk)], xv)
        pltpu.sync_copy(p0_hbm.at[pl.ds(base, chunk)], i0)
        pltpu.sync_copy(p1_hbm.at[pl.ds(base, chunk)], i1)
        pltpu.sync_copy(g1_hbm.at[pl.ds(base, chunk)], gb0)
        pltpu.sync_copy(g2_hbm.at[pl.ds(base, chunk)], gb1)
        d1 = pltpu.async_copy(xv, xs_hbm.at[i0], sem)
        d2 = pltpu.async_copy(xv, xs_hbm.at[i1], sem)
        d3 = pltpu.async_copy(gb0, gs_hbm.at[i0], sem)
        d4 = pltpu.async_copy(gb1, gs_hbm.at[i1], sem)
        d1.wait()
        d2.wait()
        d3.wait()
        d4.wait()

    return k(xf, pos0, pos1, g1b, g2b)


def _ffn_kernel(sp_ref, xs_ref, gs_ref, w1_any, b1_ref, w2_any, b2_ref, y_ref,
                w1buf, w2buf, sem):
    j = pl.program_id(0)
    nt = _NTILES
    te = sp_ref[j]
    slot = sp_ref[nt + j]
    first = sp_ref[2 * nt + j]
    nxt = sp_ref[3 * nt + j]
    active = j * _BT < sp_ref[4 * nt]

    def w1copy(e, s):
        return pltpu.make_async_copy(w1_any.at[e], w1buf.at[s], sem)

    def w2copy(e, s):
        return pltpu.make_async_copy(w2_any.at[e], w2buf.at[s], sem)

    @pl.when(active & (j == 0))
    def _start_first():
        w1copy(te, slot).start()
        w2copy(te, slot).start()

    @pl.when(active & (first == 1))
    def _wait_current():
        w1copy(te, slot).wait()
        w2copy(te, slot).wait()

    @pl.when(active & (first == 1) & (nxt != te))
    def _prefetch_next():
        w1copy(nxt, 1 - slot).start()
        w2copy(nxt, 1 - slot).start()

    @pl.when(active)
    def _compute():
        h = jnp.dot(xs_ref[...], w1buf[slot],
                    preferred_element_type=jnp.float32)
        h = jnp.maximum(h + b1_ref[0], 0.0)
        y = jnp.dot(h, w2buf[slot], preferred_element_type=jnp.float32)
        y_ref[...] = (y + b2_ref[0]) * gs_ref[:, 0:1]


def _run_ffn(sp, xs, gs, W1, b1, W2, b2):
    e, d, dff = W1.shape
    grid_spec = pltpu.PrefetchScalarGridSpec(
        num_scalar_prefetch=1,
        grid=(_NTILES,),
        in_specs=[
            pl.BlockSpec((_BT, d), lambda j, sp: (j, 0)),
            pl.BlockSpec((_BT, _GW), lambda j, sp: (j, 0)),
            pl.BlockSpec(memory_space=pl.ANY),
            pl.BlockSpec((1, 1, dff), lambda j, sp: (sp[j], 0, 0)),
            pl.BlockSpec(memory_space=pl.ANY),
            pl.BlockSpec((1, 1, d), lambda j, sp: (sp[j], 0, 0)),
        ],
        out_specs=pl.BlockSpec((_BT, d), lambda j, sp: (j, 0)),
        scratch_shapes=[
            pltpu.VMEM((2, d, dff), jnp.float32),
            pltpu.VMEM((2, dff, d), jnp.float32),
            pltpu.SemaphoreType.DMA,
        ],
    )
    return pl.pallas_call(
        _ffn_kernel,
        grid_spec=grid_spec,
        out_shape=jax.ShapeDtypeStruct((_NSLOT, d), jnp.float32),
    )(sp, xs, gs, W1, b1.reshape(e, 1, dff), W2, b2.reshape(e, 1, d))


def _combine_sc(y, pos0, pos1):
    nslot, d = y.shape
    n = pos0.shape[0]
    chunk = n // _NW
    half = chunk // 2
    mesh = plsc.VectorSubcoreMesh(core_axis_name="c", subcore_axis_name="s")

    @functools.partial(
        pl.kernel, mesh=mesh,
        out_type=jax.ShapeDtypeStruct((n, d), jnp.float32),
        scratch_types=[
            pltpu.VMEM((half, d), jnp.float32),
            pltpu.VMEM((half, d), jnp.float32),
            pltpu.VMEM((half,), jnp.int32),
            pltpu.VMEM((half,), jnp.int32),
            pltpu.SemaphoreType.DMA,
        ],
    )
    def k(y_hbm, p0_hbm, p1_hbm, out_hbm, v0, v1, i0, i1, sem):
        wid = lax.axis_index("s") * 2 + lax.axis_index("c")
        base = wid * chunk
        for hh in range(2):
            hbase = base + hh * half
            pltpu.sync_copy(p0_hbm.at[pl.ds(hbase, half)], i0)
            pltpu.sync_copy(p1_hbm.at[pl.ds(hbase, half)], i1)
            da = pltpu.async_copy(y_hbm.at[i0], v0, sem)
            db = pltpu.async_copy(y_hbm.at[i1], v1, sem)
            da.wait()
            db.wait()

            def body(r, _):
                for c in range(d // _LANES):
                    sl = pl.ds(c * _LANES, _LANES)
                    v0[r, sl] = v0[r, sl] + v1[r, sl]
                return 0

            lax.fori_loop(0, half, body, 0)
            pltpu.sync_copy(v0, out_hbm.at[pl.ds(hbase, half)])

    return k(y, pos0, pos1)


def kernel(x, W1, b1, W2, b2, Wg, bg):
    B, N, D = x.shape
    E, _, DFF = W1.shape
    xf = x.reshape(N, D)

    pos0, pos1, g1b, g2b, te, ent = _run_router(xf, Wg, bg)
    pos0 = pos0.reshape(N)
    pos1 = pos1.reshape(N)
    xs, gs = _scatter_sc(xf, pos0, pos1, g1b, g2b)
    y = _run_ffn(te.reshape(4 * _NTILES + 1), xs, gs, W1, b1, W2, b2)
    out = _combine_sc(y, pos0, pos1)
    return out.reshape(B, N, D), ent[0, 0]


# speedup vs baseline: 1.6456x; 1.0065x over previous
import functools

import jax
import jax.numpy as jnp
from jax import lax
from jax.experimental import pallas as pl
from jax.experimental.pallas import tpu as pltpu
from jax.experimental.pallas import tpu_sc as plsc

_BT = 256
_NSLOT = 6144
_NTILES = _NSLOT // _BT
_NW = 32
_LANES = 16
_GW = 128


def _router_kernel(x_ref, wg_ref, bg_ref,
                   pos0_ref, pos1_ref, g1b_ref, g2b_ref, te_ref, ent_ref):
    n, _ = x_ref.shape
    ne = wg_ref.shape[1]
    logits = jnp.dot(x_ref[...], wg_ref[...],
                     preferred_element_type=jnp.float32) + bg_ref[...]
    idx = lax.broadcasted_iota(jnp.int32, (n, ne), 1)
    m1 = jnp.max(logits, axis=1, keepdims=True)
    i1 = jnp.min(jnp.where(logits == m1, idx, ne), axis=1, keepdims=True)
    oh1 = (idx == i1).astype(jnp.float32)
    masked = jnp.where(idx == i1, -jnp.inf, logits)
    m2 = jnp.max(masked, axis=1, keepdims=True)
    i2 = jnp.min(jnp.where(masked == m2, idx, ne), axis=1, keepdims=True)
    oh2 = (idx == i2).astype(jnp.float32)

    z = jnp.exp(m2 - m1)
    denom = 1.0 + z
    g1 = 1.0 / denom
    g2 = z / denom
    g1b_ref[...] = g1
    g2b_ref[...] = g2
    ent_tok = -(g1 * jnp.log(jnp.clip(g1, 1e-8, None))
                + g2 * jnp.log(jnp.clip(g2, 1e-8, None)))
    ent_ref[...] = (jnp.sum(ent_tok) / n).reshape(1, 1)

    ch = 256
    ri = lax.broadcasted_iota(jnp.int32, (ch, ch), 0)
    ci = lax.broadcasted_iota(jnp.int32, (ch, ch), 1)
    ltri = (ci < ri).astype(jnp.float32)
    ohcat = jnp.concatenate([oh1, oh2], axis=1)
    carry = jnp.zeros((1, 2 * ne), jnp.float32)
    chunks = []
    for c in range(n // ch):
        ohc = ohcat[c * ch:(c + 1) * ch, :]
        chunks.append(jnp.dot(ltri, ohc, preferred_element_type=jnp.float32)
                      + carry)
        carry = carry + jnp.sum(ohc, axis=0, keepdims=True)
    csum = jnp.concatenate(chunks, axis=0)
    c0 = csum[:, :ne]
    c1 = csum[:, ne:]

    cnt0 = jnp.sum(oh1, axis=0, keepdims=True)
    cnt1 = jnp.sum(oh2, axis=0, keepdims=True)
    counts = cnt0 + cnt1
    pcnt = jnp.floor((counts + (_BT - 1)) / _BT) * _BT
    eri = lax.broadcasted_iota(jnp.int32, (ne, ne), 0)
    eci = lax.broadcasted_iota(jnp.int32, (ne, ne), 1)
    estri = (eri < eci).astype(jnp.float32)
    offs = jnp.dot(pcnt, estri, preferred_element_type=jnp.float32)

    pos0 = jnp.sum(oh1 * (offs + c0), axis=1, keepdims=True)
    pos1 = jnp.sum(oh2 * (offs + cnt0 + c1), axis=1, keepdims=True)
    pos0_ref[...] = pos0.astype(jnp.int32)
    pos1_ref[...] = pos1.astype(jnp.int32)

    nt = _NTILES
    jrow = (lax.broadcasted_iota(jnp.int32, (nt, ne), 0) * _BT).astype(jnp.float32)
    offs_b = jnp.broadcast_to(offs, (nt, ne))
    te = jnp.sum((offs_b <= jrow).astype(jnp.float32), axis=1, keepdims=True) - 1.0

    te_shift = jnp.concatenate([jnp.full((1, 1), -1.0, jnp.float32),
                                te[:nt - 1]], axis=0)
    bnd = (te != te_shift).astype(jnp.float32)
    tri = lax.broadcasted_iota(jnp.int32, (nt, nt), 1) <= \
        lax.broadcasted_iota(jnp.int32, (nt, nt), 0)
    cumb = jnp.dot(tri.astype(jnp.float32), bnd,
                   preferred_element_type=jnp.float32)
    slot = (cumb - 1.0) - 2.0 * jnp.floor((cumb - 1.0) / 2.0)
    active_row = (pcnt > 0).astype(jnp.float32)
    m = (eci > eri) & (jnp.broadcast_to(active_row, (ne, ne)) > 0)
    nxt_e = jnp.min(jnp.where(m, eci.astype(jnp.float32), 99.0),
                    axis=1, keepdims=True)
    te_oh = (te == lax.broadcasted_iota(jnp.int32, (nt, ne), 1)
             .astype(jnp.float32)).astype(jnp.float32)
    nxt = jnp.dot(te_oh, nxt_e, preferred_element_type=jnp.float32)
    nxt = jnp.where(nxt == 99.0, te, nxt)

    total = jnp.sum(pcnt).reshape(1, 1)
    spx = jnp.concatenate([te, slot, bnd, nxt, total], axis=0)
    te_ref[...] = spx.astype(jnp.int32)


def _run_router(xf, Wg, bg):
    n, d = xf.shape
    ne = Wg.shape[1]
    return pl.pallas_call(
        _router_kernel,
        out_shape=[
            jax.ShapeDtypeStruct((n, 1), jnp.int32),
            jax.ShapeDtypeStruct((n, 1), jnp.int32),
            jax.ShapeDtypeStruct((n, 1), jnp.float32),
            jax.ShapeDtypeStruct((n, 1), jnp.float32),
            jax.ShapeDtypeStruct((4 * _NTILES + 1, 1), jnp.int32),
            jax.ShapeDtypeStruct((1, 1), jnp.float32),
        ],
    )(xf, Wg, bg.reshape(1, ne))


def _scatter_sc(xf, pos0, pos1):
    n, d = xf.shape
    chunk = n // _NW
    mesh = plsc.VectorSubcoreMesh(core_axis_name="c", subcore_axis_name="s")

    @functools.partial(
        pl.kernel, mesh=mesh,
        out_type=jax.ShapeDtypeStruct((_NSLOT, d), jnp.float32),
        scratch_types=[
            pltpu.VMEM((chunk, d), jnp.float32),
            pltpu.VMEM((chunk,), jnp.int32),
            pltpu.VMEM((chunk,), jnp.int32),
            pltpu.SemaphoreType.DMA,
        ],
    )
    def k(x_hbm, p0_hbm, p1_hbm, xs_hbm, xv, i0, i1, sem):
        wid = lax.axis_index("s") * 2 + lax.axis_index("c")
        base = wid * chunk
        pltpu.sync_copy(x_hbm.at[pl.ds(base, chunk)], xv)
        pltpu.sync_copy(p0_hbm.at[pl.ds(base, chunk)], i0)
        pltpu.sync_copy(p1_hbm.at[pl.ds(base, chunk)], i1)
        d1 = pltpu.async_copy(xv, xs_hbm.at[i0], sem)
        d2 = pltpu.async_copy(xv, xs_hbm.at[i1], sem)
        d1.wait()
        d2.wait()

    return k(xf, pos0, pos1)


def _ffn_kernel(sp_ref, xs_ref, w1_any, b1_ref, w2_any, b2_ref, y_ref,
                w1buf, w2buf, sem):
    j = pl.program_id(0)
    nt = _NTILES
    te = sp_ref[j]
    slot = sp_ref[nt + j]
    first = sp_ref[2 * nt + j]
    nxt = sp_ref[3 * nt + j]
    active = j * _BT < sp_ref[4 * nt]

    def w1copy(e, s):
        return pltpu.make_async_copy(w1_any.at[e], w1buf.at[s], sem)

    def w2copy(e, s):
        return pltpu.make_async_copy(w2_any.at[e], w2buf.at[s], sem)

    @pl.when(active & (j == 0))
    def _start_first():
        w1copy(te, slot).start()
        w2copy(te, slot).start()

    @pl.when(active & (first == 1))
    def _wait_current():
        w1copy(te, slot).wait()
        w2copy(te, slot).wait()

    @pl.when(active & (first == 1) & (nxt != te))
    def _prefetch_next():
        w1copy(nxt, 1 - slot).start()
        w2copy(nxt, 1 - slot).start()

    @pl.when(active)
    def _compute():
        h = jnp.dot(xs_ref[...], w1buf[slot],
                    preferred_element_type=jnp.float32)
        h = jnp.maximum(h + b1_ref[0], 0.0)
        y = jnp.dot(h, w2buf[slot], preferred_element_type=jnp.float32)
        y_ref[...] = y + b2_ref[0]


def _run_ffn(sp, xs, W1, b1, W2, b2):
    e, d, dff = W1.shape
    grid_spec = pltpu.PrefetchScalarGridSpec(
        num_scalar_prefetch=1,
        grid=(_NTILES,),
        in_specs=[
            pl.BlockSpec((_BT, d), lambda j, sp: (j, 0)),
            pl.BlockSpec(memory_space=pl.ANY),
            pl.BlockSpec((1, 1, dff), lambda j, sp: (sp[j], 0, 0)),
            pl.BlockSpec(memory_space=pl.ANY),
            pl.BlockSpec((1, 1, d), lambda j, sp: (sp[j], 0, 0)),
        ],
        out_specs=pl.BlockSpec((_BT, d), lambda j, sp: (j, 0)),
        scratch_shapes=[
            pltpu.VMEM((2, d, dff), jnp.float32),
            pltpu.VMEM((2, dff, d), jnp.float32),
            pltpu.SemaphoreType.DMA,
        ],
    )
    return pl.pallas_call(
        _ffn_kernel,
        grid_spec=grid_spec,
        out_shape=jax.ShapeDtypeStruct((_NSLOT, d), jnp.float32),
    )(sp, xs, W1, b1.reshape(e, 1, dff), W2, b2.reshape(e, 1, d))


def _combine_sc(y, pos0, pos1, g1, g2):
    nslot, d = y.shape
    n = pos0.shape[0]
    chunk = n // _NW
    half = chunk // 2
    mesh = plsc.VectorSubcoreMesh(core_axis_name="c", subcore_axis_name="s")

    @functools.partial(
        pl.kernel, mesh=mesh,
        out_type=jax.ShapeDtypeStruct((n, d), jnp.float32),
        scratch_types=[
            pltpu.VMEM((half, d), jnp.float32),
            pltpu.VMEM((half, d), jnp.float32),
            pltpu.VMEM((half,), jnp.int32),
            pltpu.VMEM((half,), jnp.int32),
            pltpu.VMEM((half,), jnp.float32),
            pltpu.VMEM((half,), jnp.float32),
            pltpu.SemaphoreType.DMA,
        ],
    )
    def k(y_hbm, p0_hbm, p1_hbm, g1_hbm, g2_hbm, out_hbm,
          v0, v1, i0, i1, gc0, gc1, sem):
        wid = lax.axis_index("s") * 2 + lax.axis_index("c")
        base = wid * chunk
        dn = lax.GatherDimensionNumbers(
            offset_dims=(), collapsed_slice_dims=(0,), start_index_map=(0,))
        for hh in range(2):
            hbase = base + hh * half
            pltpu.sync_copy(p0_hbm.at[pl.ds(hbase, half)], i0)
            pltpu.sync_copy(p1_hbm.at[pl.ds(hbase, half)], i1)
            pltpu.sync_copy(g1_hbm.at[pl.ds(hbase, half)], gc0)
            pltpu.sync_copy(g2_hbm.at[pl.ds(hbase, half)], gc1)
            da = pltpu.async_copy(y_hbm.at[i0], v0, sem)
            db = pltpu.async_copy(y_hbm.at[i1], v1, sem)
            da.wait()
            db.wait()

            def body(r, _):
                grp = pl.ds((r // _LANES) * _LANES, _LANES)
                bidx = jnp.full((_LANES,), r % _LANES, jnp.int32)
                s0 = lax.gather(gc0[grp], bidx[:, None], dn, (1,),
                                mode=lax.GatherScatterMode.PROMISE_IN_BOUNDS)
                s1 = lax.gather(gc1[grp], bidx[:, None], dn, (1,),
                                mode=lax.GatherScatterMode.PROMISE_IN_BOUNDS)
                for c in range(d // _LANES):
                    sl = pl.ds(c * _LANES, _LANES)
                    v0[r, sl] = v0[r, sl] * s0 + v1[r, sl] * s1
                return 0

            lax.fori_loop(0, half, body, 0)
            pltpu.sync_copy(v0, out_hbm.at[pl.ds(hbase, half)])

    return k(y, pos0, pos1, g1, g2)


def kernel(x, W1, b1, W2, b2, Wg, bg):
    B, N, D = x.shape
    E, _, DFF = W1.shape
    xf = x.reshape(N, D)

    pos0, pos1, g1, g2, te, ent = _run_router(xf, Wg, bg)
    pos0 = pos0.reshape(N)
    pos1 = pos1.reshape(N)
    xs = _scatter_sc(xf, pos0, pos1)
    y = _run_ffn(te.reshape(4 * _NTILES + 1), xs, W1, b1, W2, b2)
    out = _combine_sc(y, pos0, pos1, g1.reshape(N), g2.reshape(N))
    return out.reshape(B, N, D), ent[0, 0]
